# baseline (speedup 1.0000x reference)
import jax
import jax.numpy as jnp
import numpy as np
from jax.experimental import pallas as pl
from jax.experimental.pallas import tpu as pltpu

_SCALE = float(2.0 ** -2.5)


def _body(x_ref, o_ref):
    xb = x_ref[...]
    xt = jnp.transpose(xb)
    s = xt.reshape(xt.shape[0] // 32, 32, xt.shape[1]).sum(axis=1) * _SCALE
    o_ref[...] = jnp.transpose(s)


def kernel(x):
    B, T = x.shape
    F = T // 32
    CB = 4096
    RB = 64
    out = pl.pallas_call(
        _body,
        grid=(B // RB, T // CB),
        in_specs=[pl.BlockSpec((RB, CB), lambda i, j: (i, j))],
        out_specs=pl.BlockSpec((RB, CB // 32), lambda i, j: (i, j)),
        out_shape=jax.ShapeDtypeStruct((B, F), jnp.float32),
        compiler_params=pltpu.CompilerParams(
            dimension_semantics=("parallel", "parallel"),
        ),
    )(x)
    return out[:, :, None]


# speedup vs baseline: 1.7416x; 1.7416x over previous
import jax
import jax.numpy as jnp
import numpy as np
from jax.experimental import pallas as pl
from jax.experimental.pallas import tpu as pltpu

_SCALE = float(2.0 ** -2.5)


def _body(x_ref, o_ref):
    xb = x_ref[...]
    xt = jnp.transpose(xb)
    s = xt.reshape(xt.shape[0] // 32, 32, xt.shape[1]).sum(axis=1) * _SCALE
    o_ref[...] = jnp.transpose(s)


def kernel(x):
    B, T = x.shape
    F = T // 32
    CB = 16384
    out = pl.pallas_call(
        _body,
        grid=(T // CB,),
        in_specs=[pl.BlockSpec((B, CB), lambda i: (0, i))],
        out_specs=pl.BlockSpec((B, CB // 32), lambda i: (0, i)),
        out_shape=jax.ShapeDtypeStruct((B, F), jnp.float32),
        compiler_params=pltpu.CompilerParams(
            dimension_semantics=("parallel",),
        ),
    )(x)
    return out[:, :, None]
